# Initial kernel scaffold; baseline (speedup 1.0000x reference)
import functools

import jax
import jax.numpy as jnp
from jax import lax
from jax.experimental import pallas as pl
from jax.experimental.pallas import tpu as pltpu
from jax.experimental.pallas import tpu_sc as plsc

_NC = 2
_NS = 16
_NW = _NC * _NS

_CHUNK = 80


def _make_aggregate(n, d, e, with_counts):
  per_w = e // _NW
  n_chunks = per_w // _CHUNK
  assert per_w * _NW == e and n_chunks * _CHUNK == per_w
  stripe = n // _NS
  assert stripe * _NS == n

  mesh = plsc.VectorSubcoreMesh(core_axis_name="c", subcore_axis_name="s")

  out_type = [jax.ShapeDtypeStruct((_NC, n, d), jnp.float32)]
  scratch = [
      pltpu.VMEM((_CHUNK,), jnp.int32),
      pltpu.VMEM((_CHUNK,), jnp.int32),
      pltpu.VMEM((_CHUNK, d), jnp.float32),
      pltpu.VMEM_SHARED((n, d), jnp.float32),
      pltpu.SemaphoreType.DMA,
  ]
  if with_counts:
    out_type.append(jax.ShapeDtypeStruct((_NC, n, 16), jnp.float32))
    scratch += [
        pltpu.VMEM((_CHUNK, 16), jnp.float32),
        pltpu.VMEM_SHARED((n, 16), jnp.float32),
    ]

  def body(x_hbm, src_hbm, dst_hbm, zeros_hbm, ones_hbm, *rest):
    if with_counts:
      (part_hbm, cnt_hbm, src_v, dst_v, rows_v, acc_sh, sem,
       ones_v, cnt_sh) = rest
    else:
      (part_hbm, src_v, dst_v, rows_v, acc_sh, sem) = rest
    c = lax.axis_index("c")
    s = lax.axis_index("s")
    wid = s * _NC + c

    row0 = s * stripe
    pltpu.sync_copy(zeros_hbm.at[pl.ds(row0, stripe)],
                    acc_sh.at[pl.ds(row0, stripe)])
    if with_counts:
      pltpu.sync_copy(zeros_hbm.at[pl.ds(row0, stripe), pl.ds(0, 16)],
                      cnt_sh.at[pl.ds(row0, stripe)])
      pltpu.sync_copy(ones_hbm, ones_v)
    plsc.subcore_barrier()

    base = wid * per_w

    def step(i, carry):
      off = pl.multiple_of(base + i * _CHUNK, 8)
      pltpu.sync_copy(src_hbm.at[pl.ds(off, _CHUNK)], src_v)
      pltpu.sync_copy(dst_hbm.at[pl.ds(off, _CHUNK)], dst_v)
      pltpu.async_copy(x_hbm.at[src_v], rows_v, sem).wait()
      pltpu.sync_copy(rows_v, acc_sh.at[dst_v], add=True)
      if with_counts:
        pltpu.sync_copy(ones_v, cnt_sh.at[dst_v], add=True)
      return carry

    lax.fori_loop(0, n_chunks, step, 0)
    plsc.subcore_barrier()

    pltpu.sync_copy(acc_sh.at[pl.ds(row0, stripe)],
                    part_hbm.at[c, pl.ds(row0, stripe)])
    if with_counts:
      pltpu.sync_copy(cnt_sh.at[pl.ds(row0, stripe)],
                      cnt_hbm.at[c, pl.ds(row0, stripe)])

  return pl.kernel(body, out_type=out_type, mesh=mesh, scratch_types=scratch)


def _dense_body(x_ref, p_ref, cnt_ref, wl_ref, wr_ref, b_ref, o_ref):
  cnt = cnt_ref[0, :, 0:1] + cnt_ref[1, :, 0:1]
  mean = (p_ref[0] + p_ref[1]) / jnp.maximum(cnt, 1.0)
  acc = jnp.dot(mean, wl_ref[...], preferred_element_type=jnp.float32)
  acc += jnp.dot(x_ref[...], wr_ref[...], preferred_element_type=jnp.float32)
  acc += b_ref[...]
  nrm2 = jnp.sum(acc * acc, axis=-1, keepdims=True)
  acc = acc * lax.rsqrt(jnp.maximum(nrm2, 1e-24))
  o_ref[...] = jnp.maximum(acc, 0.0)


def _make_dense(n, d, rows):
  grid = (n // rows,)
  assert grid[0] * rows == n
  return pl.pallas_call(
      _dense_body,
      grid=grid,
      in_specs=[
          pl.BlockSpec((rows, d), lambda i: (i, 0)),
          pl.BlockSpec((_NC, rows, d), lambda i: (0, i, 0)),
          pl.BlockSpec((_NC, rows, 16), lambda i: (0, i, 0)),
          pl.BlockSpec((d, d), lambda i: (0, 0)),
          pl.BlockSpec((d, d), lambda i: (0, 0)),
          pl.BlockSpec((1, d), lambda i: (0, 0)),
      ],
      out_specs=pl.BlockSpec((rows, d), lambda i: (i, 0)),
      out_shape=jax.ShapeDtypeStruct((n, d), jnp.float32),
  )


def kernel(x, edge_index, Wl0, Wr0, b0, Wl1, Wr1, b1, Wl2, Wr2, b2):
  n, d = x.shape
  e = edge_index.shape[1]
  src = edge_index[0]
  dst = edge_index[1]
  zeros = jnp.zeros((n, d), jnp.float32)
  ones = jnp.ones((_CHUNK, 16), jnp.float32)

  agg0 = _make_aggregate(n, d, e, with_counts=True)
  agg = _make_aggregate(n, d, e, with_counts=False)
  dense = _make_dense(n, d, rows=2000)

  p, cnt = agg0(x, src, dst, zeros, ones)
  h = dense(x, p, cnt, Wl0, Wr0, b0.reshape(1, d))
  p = agg(h, src, dst, zeros, ones)
  h = dense(h, p, cnt, Wl1, Wr1, b1.reshape(1, d))
  p = agg(h, src, dst, zeros, ones)
  h = dense(h, p, cnt, Wl2, Wr2, b2.reshape(1, d))
  return h


# speedup vs baseline: 4.7869x; 4.7869x over previous
import jax
import jax.numpy as jnp
from jax import lax
from jax.experimental import pallas as pl
from jax.experimental.pallas import tpu as pltpu
from jax.experimental.pallas import tpu_sc as plsc

_NC = 2
_NS = 16
_NW = _NC * _NS

_CHUNK = 80


def _splits(n):
  stripe = (n // _NS) // 8 * 8
  stripe_last = n - stripe * (_NS - 1)
  assert stripe > 0 and stripe_last > 0 and stripe_last % 8 == 0
  return stripe, stripe_last


def _mk_stripe_loops(s, stripe, stripe_last):
  def for_stripe_chunks(fn):
    def piece(row0, nrows):
      full, rem = divmod(nrows, _CHUNK)

      def chunk(j, carry):
        fn(pl.multiple_of(row0 + j * _CHUNK, 8), _CHUNK)
        return carry

      lax.fori_loop(0, full, chunk, 0)
      if rem:
        fn(pl.multiple_of(row0 + full * _CHUNK, 8), rem)

    @pl.when(s < _NS - 1)
    def _():
      piece(pl.multiple_of(s * stripe, 8), stripe)

    @pl.when(s == _NS - 1)
    def _():
      piece((_NS - 1) * stripe, stripe_last)

  return for_stripe_chunks


def _make_aggregate(n, d, e):
  per_w = e // _NW
  n_chunks = per_w // _CHUNK
  assert per_w * _NW == e and n_chunks * _CHUNK == per_w
  stripe, stripe_last = _splits(n)

  mesh = plsc.VectorSubcoreMesh(core_axis_name="c", subcore_axis_name="s")

  def body(x_hbm, src_hbm, dst_hbm, zeros_hbm, part_hbm,
           src_v, dst_v, rows_v, acc_sh, sem):
    c = lax.axis_index("c")
    s = lax.axis_index("s")
    wid = s * _NC + c
    for_stripe_chunks = _mk_stripe_loops(s, stripe, stripe_last)

    pltpu.sync_copy(zeros_hbm.at[pl.ds(0, _CHUNK)], rows_v)
    for_stripe_chunks(
        lambda r, nr: pltpu.sync_copy(rows_v.at[pl.ds(0, nr)],
                                      acc_sh.at[pl.ds(r, nr)]))
    plsc.subcore_barrier()

    base = wid * per_w

    def step(i, carry):
      off = pl.multiple_of(base + i * _CHUNK, 8)
      pltpu.sync_copy(src_hbm.at[pl.ds(off, _CHUNK)], src_v)
      pltpu.sync_copy(dst_hbm.at[pl.ds(off, _CHUNK)], dst_v)
      pltpu.async_copy(x_hbm.at[src_v], rows_v, sem).wait()
      pltpu.sync_copy(rows_v, acc_sh.at[dst_v], add=True)
      return carry

    lax.fori_loop(0, n_chunks, step, 0)
    plsc.subcore_barrier()

    def write_chunk(r, nrows):
      ro = pl.multiple_of(c * n + r, 8)
      pltpu.sync_copy(acc_sh.at[pl.ds(r, nrows)], rows_v.at[pl.ds(0, nrows)])
      pltpu.sync_copy(rows_v.at[pl.ds(0, nrows)],
                      part_hbm.at[pl.ds(ro, nrows)])

    for_stripe_chunks(write_chunk)

  return pl.kernel(
      body,
      out_type=[jax.ShapeDtypeStruct((_NC * n, d), jnp.float32)],
      mesh=mesh,
      scratch_types=[
          pltpu.VMEM((_CHUNK,), jnp.int32),
          pltpu.VMEM((_CHUNK,), jnp.int32),
          pltpu.VMEM((_CHUNK, d), jnp.float32),
          pltpu.VMEM_SHARED((n, d), jnp.float32),
          pltpu.SemaphoreType.DMA,
      ],
  )


def _make_counts(n, d, e):
  per_w = e // _NW
  n_chunks = per_w // _CHUNK
  assert per_w * _NW == e and n_chunks * _CHUNK == per_w
  stripe, stripe_last = _splits(n)

  mesh = plsc.VectorSubcoreMesh(core_axis_name="c", subcore_axis_name="s")

  def body(dst_hbm, zeros_hbm, ones_hbm, cnt_hbm,
           dst_v, ones_v, stage_v, acc_sh, sem):
    c = lax.axis_index("c")
    s = lax.axis_index("s")
    wid = s * _NC + c
    for_stripe_chunks = _mk_stripe_loops(s, stripe, stripe_last)

    pltpu.sync_copy(zeros_hbm.at[pl.ds(0, _CHUNK)], stage_v)
    pltpu.sync_copy(ones_hbm, ones_v)
    for_stripe_chunks(
        lambda r, nr: pltpu.sync_copy(stage_v.at[pl.ds(0, nr)],
                                      acc_sh.at[pl.ds(r, nr)]))
    plsc.subcore_barrier()

    base = wid * per_w

    def step(i, carry):
      off = pl.multiple_of(base + i * _CHUNK, 8)
      pltpu.sync_copy(dst_hbm.at[pl.ds(off, _CHUNK)], dst_v)
      pltpu.sync_copy(ones_v, acc_sh.at[dst_v], add=True)
      return carry

    lax.fori_loop(0, n_chunks, step, 0)
    plsc.subcore_barrier()

    def write_chunk(r, nrows):
      ro = pl.multiple_of(c * n + r, 8)
      pltpu.sync_copy(acc_sh.at[pl.ds(r, nrows)], stage_v.at[pl.ds(0, nrows)])
      pltpu.sync_copy(stage_v.at[pl.ds(0, nrows)],
                      cnt_hbm.at[pl.ds(ro, nrows)])

    for_stripe_chunks(write_chunk)

  return pl.kernel(
      body,
      out_type=[jax.ShapeDtypeStruct((_NC * n, d), jnp.float32)],
      mesh=mesh,
      scratch_types=[
          pltpu.VMEM((_CHUNK,), jnp.int32),
          pltpu.VMEM((_CHUNK, d), jnp.float32),
          pltpu.VMEM((_CHUNK, d), jnp.float32),
          pltpu.VMEM_SHARED((n, d), jnp.float32),
          pltpu.SemaphoreType.DMA,
      ],
  )


def _dense_body(x_ref, p_ref, cnt_ref, wl_ref, wr_ref, b_ref, o_ref):
  cnt = cnt_ref[0, :, 0:1] + cnt_ref[1, :, 0:1]
  mean = (p_ref[0] + p_ref[1]) / jnp.maximum(cnt, 1.0)
  acc = jnp.dot(mean, wl_ref[...], preferred_element_type=jnp.float32)
  acc += jnp.dot(x_ref[...], wr_ref[...], preferred_element_type=jnp.float32)
  acc += b_ref[...]
  nrm2 = jnp.sum(acc * acc, axis=-1, keepdims=True)
  acc = acc * lax.rsqrt(jnp.maximum(nrm2, 1e-24))
  o_ref[...] = jnp.maximum(acc, 0.0)


def _make_dense(n, d, rows):
  grid = (n // rows,)
  assert grid[0] * rows == n
  return pl.pallas_call(
      _dense_body,
      grid=grid,
      in_specs=[
          pl.BlockSpec((rows, d), lambda i: (i, 0)),
          pl.BlockSpec((_NC, rows, d), lambda i: (0, i, 0)),
          pl.BlockSpec((_NC, rows, d), lambda i: (0, i, 0)),
          pl.BlockSpec((d, d), lambda i: (0, 0)),
          pl.BlockSpec((d, d), lambda i: (0, 0)),
          pl.BlockSpec((1, d), lambda i: (0, 0)),
      ],
      out_specs=pl.BlockSpec((rows, d), lambda i: (i, 0)),
      out_shape=jax.ShapeDtypeStruct((n, d), jnp.float32),
  )


def kernel(x, edge_index, Wl0, Wr0, b0, Wl1, Wr1, b1, Wl2, Wr2, b2):
  n, d = x.shape
  e = edge_index.shape[1]
  src = edge_index[0]
  dst = edge_index[1]
  zeros = jnp.zeros((n, d), jnp.float32)
  ones = jnp.ones((_CHUNK, d), jnp.float32)

  agg = _make_aggregate(n, d, e)
  counts = _make_counts(n, d, e)
  dense = _make_dense(n, d, rows=2000)

  (cnt,) = counts(dst, zeros, ones)
  cnt = cnt.reshape(_NC, n, d)
  (p,) = agg(x, src, dst, zeros)
  h = dense(x, p.reshape(_NC, n, d), cnt, Wl0, Wr0, b0.reshape(1, d))
  (p,) = agg(h, src, dst, zeros)
  h = dense(h, p.reshape(_NC, n, d), cnt, Wl1, Wr1, b1.reshape(1, d))
  (p,) = agg(h, src, dst, zeros)
  h = dense(h, p.reshape(_NC, n, d), cnt, Wl2, Wr2, b2.reshape(1, d))
  return h
